# Initial kernel scaffold; baseline (speedup 1.0000x reference)
import jax
import jax.numpy as jnp
from jax.experimental import pallas as pl


def kernel(inputs, kernel0, kernel1, scaling, bias, rows0, cols0, rows1, cols1):
    raise NotImplementedError("write your pallas kernel here")


# speedup vs baseline: 1.1089x; 1.1089x over previous
import functools

import jax
import jax.numpy as jnp
from jax import lax
from jax.experimental import pallas as pl
from jax.experimental.pallas import tpu as pltpu
from jax.experimental.pallas import tpu_sc as plsc

N = 4096
NNZ = 16777
L = 16
NNZP = 16896
G = NNZP // L
NW = 32
ROWS_PER_W = N // NW
R = 4
CHUNKS = ROWS_PER_W // R


def _body(x_hbm, p0_hbm, v0_hbm, p1_hbm, v1_hbm, bias_hbm, scal_hbm, out_hbm,
          p0, v0, p1, v1, bias_v, scal_v, xbuf, hbuf):
    wid = lax.axis_index("s") * 2 + lax.axis_index("c")
    row_base = wid * ROWS_PER_W

    pltpu.sync_copy(p0_hbm, p0)
    pltpu.sync_copy(v0_hbm, v0)
    pltpu.sync_copy(p1_hbm, p1)
    pltpu.sync_copy(v1_hbm, v1)
    pltpu.sync_copy(bias_hbm, bias_v)
    pltpu.sync_copy(scal_hbm, scal_v)

    scal = scal_v[pl.ds(0, L)]
    zero16 = jnp.zeros((L,), jnp.float32)

    def make_layer(src, dst, pk_ref, vv_ref):
        def layer(g, _):
            pk = pk_ref[pl.ds(g * L, L)]
            vv = vv_ref[pl.ds(g * L, L)]
            ir = jnp.right_shift(pk, 12)
            ic = jnp.bitwise_and(pk, 4095)
            for j in range(R):
                gath = plsc.load_gather(src, [ir + (j * N)])
                plsc.addupdate_scatter(dst, [ic + (j * N)], gath * vv)
            return 0
        return layer

    def chunk_body(ci, _):
        base = row_base + ci * R

        pltpu.sync_copy(x_hbm.at[pl.ds(base * N, R * N)], xbuf)

        def zero_h(g, _):
            hbuf[pl.ds(g * L, L)] = zero16
            return 0
        lax.fori_loop(0, R * N // L, zero_h, 0, unroll=8)

        lax.fori_loop(0, G, make_layer(xbuf, hbuf, p0, v0), 0, unroll=4)

        def zero_x(g, _):
            xbuf[pl.ds(g * L, L)] = zero16
            return 0
        lax.fori_loop(0, R * N // L, zero_x, 0, unroll=8)

        lax.fori_loop(0, G, make_layer(hbuf, xbuf, p1, v1), 0, unroll=4)

        def epi(g, _):
            b = bias_v[pl.ds(g * L, L)]
            for j in range(R):
                acc = xbuf[pl.ds(j * N + g * L, L)]
                xbuf[pl.ds(j * N + g * L, L)] = jnp.maximum(acc * scal + b, 0.0)
            return 0
        lax.fori_loop(0, N // L, epi, 0, unroll=2)

        pltpu.sync_copy(xbuf, out_hbm.at[pl.ds(base * N, R * N)])
        return 0

    lax.fori_loop(0, CHUNKS, chunk_body, 0)


def kernel(inputs, kernel0, kernel1, scaling, bias, rows0, cols0, rows1, cols1):
    pad = NNZP - NNZ
    p0 = jnp.concatenate([rows0 * N + cols0, jnp.zeros((pad,), jnp.int32)])
    v0 = jnp.concatenate([kernel0, jnp.zeros((pad,), jnp.float32)])
    p1 = jnp.concatenate([rows1 * N + cols1, jnp.zeros((pad,), jnp.int32)])
    v1 = jnp.concatenate([kernel1, jnp.zeros((pad,), jnp.float32)])
    scal16 = jnp.broadcast_to(scaling, (L,)).astype(jnp.float32)
    x_flat = inputs.reshape(N * N)

    mesh = plsc.VectorSubcoreMesh(core_axis_name="c", subcore_axis_name="s")
    f = pl.kernel(
        _body,
        out_type=jax.ShapeDtypeStruct((N * N,), jnp.float32),
        mesh=mesh,
        compiler_params=pltpu.CompilerParams(needs_layout_passes=False),
        scratch_types=[
            pltpu.VMEM((NNZP,), jnp.int32),
            pltpu.VMEM((NNZP,), jnp.float32),
            pltpu.VMEM((NNZP,), jnp.int32),
            pltpu.VMEM((NNZP,), jnp.float32),
            pltpu.VMEM((N,), jnp.float32),
            pltpu.VMEM((L,), jnp.float32),
            pltpu.VMEM((R * N,), jnp.float32),
            pltpu.VMEM((R * N,), jnp.float32),
        ],
    )
    out_flat = f(x_flat, p0, v0, p1, v1, bias, scal16)
    return out_flat.reshape(N, N)


# speedup vs baseline: 2.9812x; 2.6884x over previous
import functools

import jax
import jax.numpy as jnp
from jax import lax
from jax.experimental import pallas as pl
from jax.experimental.pallas import tpu as pltpu
from jax.experimental.pallas import tpu_sc as plsc

N = 4096
NNZ = 16777
L = 16
NNZP = 16896
G = NNZP // L
NW = 32
ROWS_PER_W = N // NW
R = 4
CHUNKS = ROWS_PER_W // R


def _body(x_hbm, p0_hbm, v0_hbm, p1_hbm, v1_hbm, bias_hbm, scal_hbm, out_hbm,
          p0, v0, p1, v1, bias_v, scal_v, xb0, xb1, hbuf,
          sin0, sin1, sout0, sout1):
    wid = lax.axis_index("s") * 2 + lax.axis_index("c")
    row_base = wid * ROWS_PER_W

    pltpu.sync_copy(p0_hbm, p0)
    pltpu.sync_copy(v0_hbm, v0)
    pltpu.sync_copy(p1_hbm, p1)
    pltpu.sync_copy(v1_hbm, v1)
    pltpu.sync_copy(bias_hbm, bias_v)
    pltpu.sync_copy(scal_hbm, scal_v)

    scal = scal_v[pl.ds(0, L)]
    zero16 = jnp.zeros((L,), jnp.float32)
    xbufs = (xb0, xb1)
    sins = (sin0, sin1)
    souts = (sout0, sout1)

    def xslice(ci):
        return x_hbm.at[pl.ds((row_base + ci * R) * N, R * N)]

    def oslice(ci):
        return out_hbm.at[pl.ds((row_base + ci * R) * N, R * N)]

    def run_layer(src, dst, pk_ref, vv_ref):
        @plsc.parallel_loop(0, G, 1, unroll=4)
        def layer(g):
            pk = pk_ref[pl.ds(g * L, L)]
            vv = vv_ref[pl.ds(g * L, L)]
            ir = jnp.right_shift(pk, 12)
            ic = jnp.bitwise_and(pk, 4095)
            for j in range(R):
                gath = plsc.load_gather(src, [ir + (j * N)])
                plsc.addupdate_scatter(dst, [ic + (j * N)], gath * vv)

    pltpu.async_copy(xslice(0), xb0, sin0)

    def pair_body(cp, _):
        for b in (0, 1):
            ci = cp * 2 + b
            xb = xbufs[b]

            pltpu.make_async_copy(xslice(ci), xb, sins[b]).wait()

            @plsc.parallel_loop(0, R * N // L, 1, unroll=8)
            def zero_h(g):
                hbuf[pl.ds(g * L, L)] = zero16

            run_layer(xb, hbuf, p0, v0)

            @pl.when(ci > 0)
            def _drain():
                pltpu.make_async_copy(xbufs[1 - b], oslice(ci - 1),
                                      souts[1 - b]).wait()

            @pl.when(ci + 1 < CHUNKS)
            def _prefetch():
                pltpu.async_copy(xslice(ci + 1), xbufs[1 - b], sins[1 - b])

            @plsc.parallel_loop(0, R * N // L, 1, unroll=8)
            def zero_x(g):
                xb[pl.ds(g * L, L)] = zero16

            run_layer(hbuf, xb, p1, v1)

            @plsc.parallel_loop(0, N // L, 1, unroll=4)
            def epi(g):
                bv = bias_v[pl.ds(g * L, L)]
                for j in range(R):
                    acc = xb[pl.ds(j * N + g * L, L)]
                    xb[pl.ds(j * N + g * L, L)] = jnp.maximum(
                        acc * scal + bv, 0.0)

            pltpu.async_copy(xb, oslice(ci), souts[b])
        return 0

    lax.fori_loop(0, CHUNKS // 2, pair_body, 0)

    pltpu.make_async_copy(xb1, oslice(CHUNKS - 1), sout1).wait()


def kernel(inputs, kernel0, kernel1, scaling, bias, rows0, cols0, rows1, cols1):
    pad = NNZP - NNZ
    p0 = jnp.concatenate([rows0 * N + cols0, jnp.zeros((pad,), jnp.int32)])
    v0 = jnp.concatenate([kernel0, jnp.zeros((pad,), jnp.float32)])
    p1 = jnp.concatenate([rows1 * N + cols1, jnp.zeros((pad,), jnp.int32)])
    v1 = jnp.concatenate([kernel1, jnp.zeros((pad,), jnp.float32)])
    scal16 = jnp.broadcast_to(scaling, (L,)).astype(jnp.float32)
    x_flat = inputs.reshape(N * N)

    mesh = plsc.VectorSubcoreMesh(core_axis_name="c", subcore_axis_name="s")
    f = pl.kernel(
        _body,
        out_type=jax.ShapeDtypeStruct((N * N,), jnp.float32),
        mesh=mesh,
        compiler_params=pltpu.CompilerParams(needs_layout_passes=False),
        scratch_types=[
            pltpu.VMEM((NNZP,), jnp.int32),
            pltpu.VMEM((NNZP,), jnp.float32),
            pltpu.VMEM((NNZP,), jnp.int32),
            pltpu.VMEM((NNZP,), jnp.float32),
            pltpu.VMEM((N,), jnp.float32),
            pltpu.VMEM((L,), jnp.float32),
            pltpu.VMEM((R * N,), jnp.float32),
            pltpu.VMEM((R * N,), jnp.float32),
            pltpu.VMEM((R * N,), jnp.float32),
            pltpu.SemaphoreType.DMA,
            pltpu.SemaphoreType.DMA,
            pltpu.SemaphoreType.DMA,
            pltpu.SemaphoreType.DMA,
        ],
    )
    out_flat = f(x_flat, p0, v0, p1, v1, bias, scal16)
    return out_flat.reshape(N, N)
